# Initial kernel scaffold; baseline (speedup 1.0000x reference)
import jax
import jax.numpy as jnp
from jax.experimental import pallas as pl


def kernel(cls, mask_coeff, proto, boxes, tgt_labels, tgt_masks, tgt_boxes):
    raise NotImplementedError("write your pallas kernel here")


# speedup vs baseline: 1.9871x; 1.9871x over previous
import jax
import jax.numpy as jnp
from jax.experimental import pallas as pl
from jax.experimental.pallas import tpu as pltpu

_B, _Q, _C, _M, _T, _H, _W, _G = 2, 300, 80, 32, 2, 128, 128, 20
_THW = _T * _H * _W
_K = 4096
_NK = _THW // _K
_COST_CLASS, _COST_BOX, _COST_DICE = 1.0, 3.0, 1.0
_DN = (((1,), (1,)), ((), ()))


def _body(coeff_ref, proto_ref, tgt_ref, cls_ref, boxes_ref, tgtbt_ref,
          labels_ref, cost_ref, match_ref,
          acc_xt, acc_st, acc_ssum, acc_spsum, acc_tsum):
    k = pl.program_id(1)

    @pl.when(k == 0)
    def _():
        acc_xt[...] = jnp.zeros_like(acc_xt)
        acc_st[...] = jnp.zeros_like(acc_st)
        acc_ssum[...] = jnp.zeros_like(acc_ssum)
        acc_spsum[...] = jnp.zeros_like(acc_spsum)
        acc_tsum[...] = jnp.zeros_like(acc_tsum)

    coeff = coeff_ref[0]
    ptile = proto_ref[0]
    ttile = tgt_ref[0]
    x = jnp.dot(coeff, ptile, preferred_element_type=jnp.float32)
    z = jnp.exp(-jnp.abs(x))
    l1p = jnp.log1p(z)
    posx = x > 0.0
    sig = jnp.where(posx, 1.0, z) / (1.0 + z)
    sp = jnp.where(posx, x + l1p, l1p)
    acc_xt[...] += jax.lax.dot_general(x, ttile, _DN,
                                       preferred_element_type=jnp.float32)
    acc_st[...] += jax.lax.dot_general(sig, ttile, _DN,
                                       preferred_element_type=jnp.float32)
    acc_ssum[...] += jnp.sum(sig, axis=1, keepdims=True)
    acc_spsum[...] += jnp.sum(sp, axis=1, keepdims=True)
    ones_row = jnp.ones((1, _K), jnp.float32)
    acc_tsum[...] += jax.lax.dot_general(ones_row, ttile, _DN,
                                         preferred_element_type=jnp.float32)

    @pl.when(k == _NK - 1)
    def _():
        xt = acc_xt[...]
        st = acc_st[...]
        ssum = acc_ssum[...]
        spsum = acc_spsum[...]
        tsum = acc_tsum[...]
        cost_bce = (spsum - xt) * (1.0 / _THW)
        cost_dice = 1.0 - (2.0 * st + 1.0) / (ssum + tsum + 1.0)

        probs = jax.nn.sigmoid(cls_ref[0])
        labels = labels_ref[0]
        cls_iota = jax.lax.broadcasted_iota(jnp.int32, (_C, _G), 0)
        onehot = (cls_iota == labels).astype(jnp.float32)
        cost_class = -jnp.dot(probs, onehot, preferred_element_type=jnp.float32)

        bx = boxes_ref[0]
        tb = tgtbt_ref[0]
        ax0, ay0, ax1, ay1 = (bx[:, i:i + 1] for i in range(4))
        bx0, by0, bx1, by1 = (tb[i:i + 1, :] for i in range(4))
        l1 = (jnp.abs(ax0 - bx0) + jnp.abs(ay0 - by0)
              + jnp.abs(ax1 - bx1) + jnp.abs(ay1 - by1))
        area_a = (ax1 - ax0) * (ay1 - ay0)
        area_b = (bx1 - bx0) * (by1 - by0)
        iw = jnp.clip(jnp.minimum(ax1, bx1) - jnp.maximum(ax0, bx0), 0.0)
        ih = jnp.clip(jnp.minimum(ay1, by1) - jnp.maximum(ay0, by0), 0.0)
        inter = iw * ih
        union = area_a + area_b - inter
        iou = inter / (union + 1e-7)
        ew = jnp.maximum(ax1, bx1) - jnp.minimum(ax0, bx0)
        eh = jnp.maximum(ay1, by1) - jnp.minimum(ay0, by0)
        enc = jnp.clip(ew, 0.0) * jnp.clip(eh, 0.0)
        giou = iou - (enc - union) / (enc + 1e-7)
        cost_bbox = l1 + (1.0 - giou)

        cost = (_COST_CLASS * cost_class
                + _COST_DICE * (cost_bce + cost_dice)
                + _COST_BOX * cost_bbox)
        cost_ref[0] = cost
        qiota = jax.lax.broadcasted_iota(jnp.int32, (_Q, _G), 0)
        cmin = jnp.min(cost, axis=0, keepdims=True)
        match_ref[0] = jnp.min(jnp.where(cost == cmin, qiota, _Q), axis=0,
                               keepdims=True)


def kernel(cls, mask_coeff, proto, boxes, tgt_labels, tgt_masks, tgt_boxes):
    proto2 = proto.reshape(_B, _M, _THW)
    tgt2 = tgt_masks.reshape(_B, _G, _THW)
    tgtbt = jnp.swapaxes(tgt_boxes, 1, 2)
    labels3 = tgt_labels.astype(jnp.int32).reshape(_B, 1, _G)
    cost, match3 = pl.pallas_call(
        _body,
        grid=(_B, _NK),
        in_specs=[
            pl.BlockSpec((1, _Q, _M), lambda b, k: (b, 0, 0)),
            pl.BlockSpec((1, _M, _K), lambda b, k: (b, 0, k)),
            pl.BlockSpec((1, _G, _K), lambda b, k: (b, 0, k)),
            pl.BlockSpec((1, _Q, _C), lambda b, k: (b, 0, 0)),
            pl.BlockSpec((1, _Q, 4), lambda b, k: (b, 0, 0)),
            pl.BlockSpec((1, 4, _G), lambda b, k: (b, 0, 0)),
            pl.BlockSpec((1, 1, _G), lambda b, k: (b, 0, 0)),
        ],
        out_specs=[
            pl.BlockSpec((1, _Q, _G), lambda b, k: (b, 0, 0)),
            pl.BlockSpec((1, 1, _G), lambda b, k: (b, 0, 0)),
        ],
        out_shape=[
            jax.ShapeDtypeStruct((_B, _Q, _G), jnp.float32),
            jax.ShapeDtypeStruct((_B, 1, _G), jnp.int32),
        ],
        scratch_shapes=[
            pltpu.VMEM((_Q, _G), jnp.float32),
            pltpu.VMEM((_Q, _G), jnp.float32),
            pltpu.VMEM((_Q, 1), jnp.float32),
            pltpu.VMEM((_Q, 1), jnp.float32),
            pltpu.VMEM((1, _G), jnp.float32),
        ],
        compiler_params=pltpu.CompilerParams(
            dimension_semantics=("parallel", "arbitrary")),
    )(mask_coeff, proto2, tgt2, cls, boxes, tgtbt, labels3)
    return cost, match3[:, 0, :]


# speedup vs baseline: 2.0495x; 1.0314x over previous
import jax
import jax.numpy as jnp
from jax.experimental import pallas as pl
from jax.experimental.pallas import tpu as pltpu

_B, _Q, _C, _M, _T, _H, _W, _G = 2, 300, 80, 32, 2, 128, 128, 20
_THW = _T * _H * _W
_K = 4096
_NK = _THW // _K
_COST_CLASS, _COST_BOX, _COST_DICE = 1.0, 3.0, 1.0
_DN = (((1,), (1,)), ((), ()))


def _body(coeff_ref, proto_ref, tgt_ref, cls_ref, boxes_ref, tgtbt_ref,
          labels_ref, cost_ref, match_ref,
          acc_xt, acc_st, acc_ssum, acc_spsum, acc_tsum):
    k = pl.program_id(1)

    @pl.when(k == 0)
    def _():
        acc_xt[...] = jnp.zeros_like(acc_xt)
        acc_st[...] = jnp.zeros_like(acc_st)
        acc_ssum[...] = jnp.zeros_like(acc_ssum)
        acc_spsum[...] = jnp.zeros_like(acc_spsum)
        acc_tsum[...] = jnp.zeros_like(acc_tsum)

    coeff = coeff_ref[0].astype(jnp.bfloat16)
    ptile = proto_ref[0].astype(jnp.bfloat16)
    ttile = tgt_ref[0].astype(jnp.bfloat16)
    x = jnp.dot(coeff, ptile, preferred_element_type=jnp.float32)
    z = jnp.exp(-jnp.abs(x))
    l1p = jnp.log1p(z)
    posx = x > 0.0
    sig = jnp.where(posx, 1.0, z) / (1.0 + z)
    sp = jnp.where(posx, x + l1p, l1p)
    acc_xt[...] += jax.lax.dot_general(x.astype(jnp.bfloat16), ttile, _DN,
                                       preferred_element_type=jnp.float32)
    acc_st[...] += jax.lax.dot_general(sig.astype(jnp.bfloat16), ttile, _DN,
                                       preferred_element_type=jnp.float32)
    acc_ssum[...] += jnp.sum(sig, axis=1, keepdims=True)
    acc_spsum[...] += jnp.sum(sp, axis=1, keepdims=True)
    ones_row = jnp.ones((1, _K), jnp.bfloat16)
    acc_tsum[...] += jax.lax.dot_general(ones_row, ttile, _DN,
                                         preferred_element_type=jnp.float32)

    @pl.when(k == _NK - 1)
    def _():
        xt = acc_xt[...]
        st = acc_st[...]
        ssum = acc_ssum[...]
        spsum = acc_spsum[...]
        tsum = acc_tsum[...]
        cost_bce = (spsum - xt) * (1.0 / _THW)
        cost_dice = 1.0 - (2.0 * st + 1.0) / (ssum + tsum + 1.0)

        probs = jax.nn.sigmoid(cls_ref[0])
        labels = labels_ref[0]
        cls_iota = jax.lax.broadcasted_iota(jnp.int32, (_C, _G), 0)
        onehot = (cls_iota == labels).astype(jnp.float32)
        cost_class = -jnp.dot(probs, onehot, preferred_element_type=jnp.float32)

        bx = boxes_ref[0]
        tb = tgtbt_ref[0]
        ax0, ay0, ax1, ay1 = (bx[:, i:i + 1] for i in range(4))
        bx0, by0, bx1, by1 = (tb[i:i + 1, :] for i in range(4))
        l1 = (jnp.abs(ax0 - bx0) + jnp.abs(ay0 - by0)
              + jnp.abs(ax1 - bx1) + jnp.abs(ay1 - by1))
        area_a = (ax1 - ax0) * (ay1 - ay0)
        area_b = (bx1 - bx0) * (by1 - by0)
        iw = jnp.clip(jnp.minimum(ax1, bx1) - jnp.maximum(ax0, bx0), 0.0)
        ih = jnp.clip(jnp.minimum(ay1, by1) - jnp.maximum(ay0, by0), 0.0)
        inter = iw * ih
        union = area_a + area_b - inter
        iou = inter / (union + 1e-7)
        ew = jnp.maximum(ax1, bx1) - jnp.minimum(ax0, bx0)
        eh = jnp.maximum(ay1, by1) - jnp.minimum(ay0, by0)
        enc = jnp.clip(ew, 0.0) * jnp.clip(eh, 0.0)
        giou = iou - (enc - union) / (enc + 1e-7)
        cost_bbox = l1 + (1.0 - giou)

        cost = (_COST_CLASS * cost_class
                + _COST_DICE * (cost_bce + cost_dice)
                + _COST_BOX * cost_bbox)
        cost_ref[0] = cost
        qiota = jax.lax.broadcasted_iota(jnp.int32, (_Q, _G), 0)
        cmin = jnp.min(cost, axis=0, keepdims=True)
        match_ref[0] = jnp.min(jnp.where(cost == cmin, qiota, _Q), axis=0,
                               keepdims=True)


def kernel(cls, mask_coeff, proto, boxes, tgt_labels, tgt_masks, tgt_boxes):
    proto2 = proto.reshape(_B, _M, _THW)
    tgt2 = tgt_masks.reshape(_B, _G, _THW)
    tgtbt = jnp.swapaxes(tgt_boxes, 1, 2)
    labels3 = tgt_labels.astype(jnp.int32).reshape(_B, 1, _G)
    cost, match3 = pl.pallas_call(
        _body,
        grid=(_B, _NK),
        in_specs=[
            pl.BlockSpec((1, _Q, _M), lambda b, k: (b, 0, 0)),
            pl.BlockSpec((1, _M, _K), lambda b, k: (b, 0, k)),
            pl.BlockSpec((1, _G, _K), lambda b, k: (b, 0, k)),
            pl.BlockSpec((1, _Q, _C), lambda b, k: (b, 0, 0)),
            pl.BlockSpec((1, _Q, 4), lambda b, k: (b, 0, 0)),
            pl.BlockSpec((1, 4, _G), lambda b, k: (b, 0, 0)),
            pl.BlockSpec((1, 1, _G), lambda b, k: (b, 0, 0)),
        ],
        out_specs=[
            pl.BlockSpec((1, _Q, _G), lambda b, k: (b, 0, 0)),
            pl.BlockSpec((1, 1, _G), lambda b, k: (b, 0, 0)),
        ],
        out_shape=[
            jax.ShapeDtypeStruct((_B, _Q, _G), jnp.float32),
            jax.ShapeDtypeStruct((_B, 1, _G), jnp.int32),
        ],
        scratch_shapes=[
            pltpu.VMEM((_Q, _G), jnp.float32),
            pltpu.VMEM((_Q, _G), jnp.float32),
            pltpu.VMEM((_Q, 1), jnp.float32),
            pltpu.VMEM((_Q, 1), jnp.float32),
            pltpu.VMEM((1, _G), jnp.float32),
        ],
        compiler_params=pltpu.CompilerParams(
            dimension_semantics=("parallel", "arbitrary")),
    )(mask_coeff, proto2, tgt2, cls, boxes, tgtbt, labels3)
    return cost, match3[:, 0, :]


# speedup vs baseline: 2.7164x; 1.3254x over previous
import jax
import jax.numpy as jnp
from jax.experimental import pallas as pl
from jax.experimental.pallas import tpu as pltpu

_B, _Q, _C, _M, _T, _H, _W, _G = 2, 300, 80, 32, 2, 128, 128, 20
_THW = _T * _H * _W
_K = 4096
_NK = _THW // _K
_G1 = _G + 1
_COST_CLASS, _COST_BOX, _COST_DICE = 1.0, 3.0, 1.0
_LN2 = 0.6931471805599453
_DN = (((1,), (1,)), ((), ()))


def _body(coeffh_ref, proto_ref, rhs_ref, cls_ref, boxes_ref, tgtbt_ref,
          labels_ref, cost_ref, match_ref, acc_x, acc_t, acc_sp, acc_ts):
    k = pl.program_id(1)

    @pl.when(k == 0)
    def _():
        acc_x[...] = jnp.zeros_like(acc_x)
        acc_t[...] = jnp.zeros_like(acc_t)
        acc_sp[...] = jnp.zeros_like(acc_sp)
        acc_ts[...] = jnp.zeros_like(acc_ts)

    coeffh = coeffh_ref[0]
    ptile = proto_ref[0]
    rtile = rhs_ref[0]
    xh = jnp.dot(coeffh, ptile, preferred_element_type=jnp.float32)
    tv = jnp.tanh(xh)
    spc = 2.0 * jnp.maximum(xh, 0.0) - jnp.log(1.0 + jnp.abs(tv))
    acc_x[...] += jax.lax.dot_general(xh.astype(jnp.bfloat16), rtile, _DN,
                                      preferred_element_type=jnp.float32)
    acc_t[...] += jax.lax.dot_general(tv.astype(jnp.bfloat16), rtile, _DN,
                                      preferred_element_type=jnp.float32)
    acc_sp[...] += jax.lax.dot_general(spc.astype(jnp.bfloat16), rtile, _DN,
                                       preferred_element_type=jnp.float32)
    ones_row = jnp.ones((1, _K), jnp.bfloat16)
    acc_ts[...] += jax.lax.dot_general(ones_row, rtile, _DN,
                                       preferred_element_type=jnp.float32)

    @pl.when(k == _NK - 1)
    def _():
        xt = 2.0 * acc_x[:, :_G]
        ttm = acc_t[...]
        tsum = acc_ts[:, :_G]
        st = 0.5 * (tsum + ttm[:, :_G])
        ssum = 0.5 * (_THW + ttm[:, _G:])
        spsum = acc_sp[:, _G:] + _THW * _LN2
        cost_bce = (spsum - xt) * (1.0 / _THW)
        cost_dice = 1.0 - (2.0 * st + 1.0) / (ssum + tsum + 1.0)

        probs = jax.nn.sigmoid(cls_ref[0])
        labels = labels_ref[0]
        cls_iota = jax.lax.broadcasted_iota(jnp.int32, (_C, _G), 0)
        onehot = (cls_iota == labels).astype(jnp.float32)
        cost_class = -jnp.dot(probs, onehot, preferred_element_type=jnp.float32)

        bx = boxes_ref[0]
        tb = tgtbt_ref[0]
        ax0, ay0, ax1, ay1 = (bx[:, i:i + 1] for i in range(4))
        bx0, by0, bx1, by1 = (tb[i:i + 1, :] for i in range(4))
        l1 = (jnp.abs(ax0 - bx0) + jnp.abs(ay0 - by0)
              + jnp.abs(ax1 - bx1) + jnp.abs(ay1 - by1))
        area_a = (ax1 - ax0) * (ay1 - ay0)
        area_b = (bx1 - bx0) * (by1 - by0)
        iw = jnp.clip(jnp.minimum(ax1, bx1) - jnp.maximum(ax0, bx0), 0.0)
        ih = jnp.clip(jnp.minimum(ay1, by1) - jnp.maximum(ay0, by0), 0.0)
        inter = iw * ih
        union = area_a + area_b - inter
        iou = inter / (union + 1e-7)
        ew = jnp.maximum(ax1, bx1) - jnp.minimum(ax0, bx0)
        eh = jnp.maximum(ay1, by1) - jnp.minimum(ay0, by0)
        enc = jnp.clip(ew, 0.0) * jnp.clip(eh, 0.0)
        giou = iou - (enc - union) / (enc + 1e-7)
        cost_bbox = l1 + (1.0 - giou)

        cost = (_COST_CLASS * cost_class
                + _COST_DICE * (cost_bce + cost_dice)
                + _COST_BOX * cost_bbox)
        cost_ref[0] = cost
        qiota = jax.lax.broadcasted_iota(jnp.int32, (_Q, _G), 0)
        cmin = jnp.min(cost, axis=0, keepdims=True)
        match_ref[0] = jnp.min(jnp.where(cost == cmin, qiota, _Q), axis=0,
                               keepdims=True)


def kernel(cls, mask_coeff, proto, boxes, tgt_labels, tgt_masks, tgt_boxes):
    coeffh = (0.5 * mask_coeff).astype(jnp.bfloat16)
    proto2 = proto.reshape(_B, _M, _THW).astype(jnp.bfloat16)
    rhs = jnp.concatenate(
        [tgt_masks.reshape(_B, _G, _THW),
         jnp.ones((_B, 1, _THW), jnp.float32)], axis=1).astype(jnp.bfloat16)
    tgtbt = jnp.swapaxes(tgt_boxes, 1, 2)
    labels3 = tgt_labels.astype(jnp.int32).reshape(_B, 1, _G)
    cost, match3 = pl.pallas_call(
        _body,
        grid=(_B, _NK),
        in_specs=[
            pl.BlockSpec((1, _Q, _M), lambda b, k: (b, 0, 0)),
            pl.BlockSpec((1, _M, _K), lambda b, k: (b, 0, k)),
            pl.BlockSpec((1, _G1, _K), lambda b, k: (b, 0, k)),
            pl.BlockSpec((1, _Q, _C), lambda b, k: (b, 0, 0)),
            pl.BlockSpec((1, _Q, 4), lambda b, k: (b, 0, 0)),
            pl.BlockSpec((1, 4, _G), lambda b, k: (b, 0, 0)),
            pl.BlockSpec((1, 1, _G), lambda b, k: (b, 0, 0)),
        ],
        out_specs=[
            pl.BlockSpec((1, _Q, _G), lambda b, k: (b, 0, 0)),
            pl.BlockSpec((1, 1, _G), lambda b, k: (b, 0, 0)),
        ],
        out_shape=[
            jax.ShapeDtypeStruct((_B, _Q, _G), jnp.float32),
            jax.ShapeDtypeStruct((_B, 1, _G), jnp.int32),
        ],
        scratch_shapes=[
            pltpu.VMEM((_Q, _G1), jnp.float32),
            pltpu.VMEM((_Q, _G1), jnp.float32),
            pltpu.VMEM((_Q, _G1), jnp.float32),
            pltpu.VMEM((1, _G1), jnp.float32),
        ],
        compiler_params=pltpu.CompilerParams(
            dimension_semantics=("parallel", "arbitrary")),
    )(coeffh, proto2, rhs, cls, boxes, tgtbt, labels3)
    return cost, match3[:, 0, :]


# speedup vs baseline: 3.0855x; 1.1358x over previous
import jax
import jax.numpy as jnp
from jax.experimental import pallas as pl
from jax.experimental.pallas import tpu as pltpu

_B, _Q, _C, _M, _T, _H, _W, _G = 2, 300, 80, 32, 2, 128, 128, 20
_THW = _T * _H * _W
_K = 4096
_NK = _THW // _K
_G1 = _G + 1
_COST_CLASS, _COST_BOX, _COST_DICE = 1.0, 3.0, 1.0
_LN2 = 0.6931471805599453
_DN = (((1,), (1,)), ((), ()))


def _body(coeffh_ref, proto_ref, tgt_ref, cls_ref, boxes_ref, tgtbt_ref,
          labels_ref, cost_ref, match_ref, acc_pt, acc_t, acc_sp, acc_ts):
    k = pl.program_id(1)

    @pl.when(k == 0)
    def _():
        acc_pt[...] = jnp.zeros_like(acc_pt)
        acc_t[...] = jnp.zeros_like(acc_t)
        acc_sp[...] = jnp.zeros_like(acc_sp)
        acc_ts[...] = jnp.zeros_like(acc_ts)

    coeffh = coeffh_ref[0]
    ptile = proto_ref[0].astype(jnp.bfloat16)
    rtile = jnp.concatenate(
        [tgt_ref[0].astype(jnp.bfloat16),
         jnp.ones((1, _K), jnp.bfloat16)], axis=0)
    xh = jnp.dot(coeffh, ptile, preferred_element_type=jnp.float32)
    tv = jnp.tanh(xh)
    spc = 2.0 * jnp.maximum(xh, 0.0) - jnp.log(1.0 + jnp.abs(tv))
    acc_pt[...] += jax.lax.dot_general(ptile, rtile, _DN,
                                       preferred_element_type=jnp.float32)
    acc_t[...] += jax.lax.dot_general(tv.astype(jnp.bfloat16), rtile, _DN,
                                      preferred_element_type=jnp.float32)
    acc_sp[...] += jax.lax.dot_general(spc.astype(jnp.bfloat16), rtile, _DN,
                                       preferred_element_type=jnp.float32)
    ones_row = jnp.ones((1, _K), jnp.bfloat16)
    acc_ts[...] += jax.lax.dot_general(ones_row, rtile, _DN,
                                       preferred_element_type=jnp.float32)

    @pl.when(k == _NK - 1)
    def _():
        xt = 2.0 * jnp.dot(coeffh_ref[0].astype(jnp.float32), acc_pt[:, :_G],
                           preferred_element_type=jnp.float32)
        ttm = acc_t[...]
        tsum = acc_ts[:, :_G]
        st = 0.5 * (tsum + ttm[:, :_G])
        ssum = 0.5 * (_THW + ttm[:, _G:])
        spsum = acc_sp[:, _G:] + _THW * _LN2
        cost_bce = (spsum - xt) * (1.0 / _THW)
        cost_dice = 1.0 - (2.0 * st + 1.0) / (ssum + tsum + 1.0)

        probs = jax.nn.sigmoid(cls_ref[0])
        labels = labels_ref[0]
        cls_iota = jax.lax.broadcasted_iota(jnp.int32, (_C, _G), 0)
        onehot = (cls_iota == labels).astype(jnp.float32)
        cost_class = -jnp.dot(probs, onehot, preferred_element_type=jnp.float32)

        bx = boxes_ref[0]
        tb = tgtbt_ref[0]
        ax0, ay0, ax1, ay1 = (bx[:, i:i + 1] for i in range(4))
        bx0, by0, bx1, by1 = (tb[i:i + 1, :] for i in range(4))
        l1 = (jnp.abs(ax0 - bx0) + jnp.abs(ay0 - by0)
              + jnp.abs(ax1 - bx1) + jnp.abs(ay1 - by1))
        area_a = (ax1 - ax0) * (ay1 - ay0)
        area_b = (bx1 - bx0) * (by1 - by0)
        iw = jnp.clip(jnp.minimum(ax1, bx1) - jnp.maximum(ax0, bx0), 0.0)
        ih = jnp.clip(jnp.minimum(ay1, by1) - jnp.maximum(ay0, by0), 0.0)
        inter = iw * ih
        union = area_a + area_b - inter
        iou = inter / (union + 1e-7)
        ew = jnp.maximum(ax1, bx1) - jnp.minimum(ax0, bx0)
        eh = jnp.maximum(ay1, by1) - jnp.minimum(ay0, by0)
        enc = jnp.clip(ew, 0.0) * jnp.clip(eh, 0.0)
        giou = iou - (enc - union) / (enc + 1e-7)
        cost_bbox = l1 + (1.0 - giou)

        cost = (_COST_CLASS * cost_class
                + _COST_DICE * (cost_bce + cost_dice)
                + _COST_BOX * cost_bbox)
        cost_ref[0] = cost
        qiota = jax.lax.broadcasted_iota(jnp.int32, (_Q, _G), 0)
        cmin = jnp.min(cost, axis=0, keepdims=True)
        match_ref[0] = jnp.min(jnp.where(cost == cmin, qiota, _Q), axis=0,
                               keepdims=True)


def kernel(cls, mask_coeff, proto, boxes, tgt_labels, tgt_masks, tgt_boxes):
    coeffh = (0.5 * mask_coeff).astype(jnp.bfloat16)
    proto2 = proto.reshape(_B, _M, _THW)
    tgt2 = tgt_masks.reshape(_B, _G, _THW)
    tgtbt = jnp.swapaxes(tgt_boxes, 1, 2)
    labels3 = tgt_labels.astype(jnp.int32).reshape(_B, 1, _G)
    cost, match3 = pl.pallas_call(
        _body,
        grid=(_B, _NK),
        in_specs=[
            pl.BlockSpec((1, _Q, _M), lambda b, k: (b, 0, 0)),
            pl.BlockSpec((1, _M, _K), lambda b, k: (b, 0, k)),
            pl.BlockSpec((1, _G, _K), lambda b, k: (b, 0, k)),
            pl.BlockSpec((1, _Q, _C), lambda b, k: (b, 0, 0)),
            pl.BlockSpec((1, _Q, 4), lambda b, k: (b, 0, 0)),
            pl.BlockSpec((1, 4, _G), lambda b, k: (b, 0, 0)),
            pl.BlockSpec((1, 1, _G), lambda b, k: (b, 0, 0)),
        ],
        out_specs=[
            pl.BlockSpec((1, _Q, _G), lambda b, k: (b, 0, 0)),
            pl.BlockSpec((1, 1, _G), lambda b, k: (b, 0, 0)),
        ],
        out_shape=[
            jax.ShapeDtypeStruct((_B, _Q, _G), jnp.float32),
            jax.ShapeDtypeStruct((_B, 1, _G), jnp.int32),
        ],
        scratch_shapes=[
            pltpu.VMEM((_M, _G1), jnp.float32),
            pltpu.VMEM((_Q, _G1), jnp.float32),
            pltpu.VMEM((_Q, _G1), jnp.float32),
            pltpu.VMEM((1, _G1), jnp.float32),
        ],
        compiler_params=pltpu.CompilerParams(
            dimension_semantics=("parallel", "arbitrary")),
    )(coeffh, proto2, tgt2, cls, boxes, tgtbt, labels3)
    return cost, match3[:, 0, :]


# speedup vs baseline: 3.5669x; 1.1560x over previous
import jax
import jax.numpy as jnp
from jax.experimental import pallas as pl
from jax.experimental.pallas import tpu as pltpu

_B, _Q, _C, _M, _T, _H, _W, _G = 2, 300, 80, 32, 2, 128, 128, 20
_THW = _T * _H * _W
_K = 4096
_NK = _THW // _K
_G1 = _G + 1
_COST_CLASS, _COST_BOX, _COST_DICE = 1.0, 3.0, 1.0
_LN2 = 0.6931471805599453
_DN = (((1,), (1,)), ((), ()))


def _body(coeffh_ref, proto_ref, tgt_ref, cls_ref, boxes_ref, tgtbt_ref,
          labels_ref, cost_ref, match_ref, acc_pt, acc_t, acc_sp, acc_ts):
    k = pl.program_id(1)

    @pl.when(k == 0)
    def _():
        acc_pt[...] = jnp.zeros_like(acc_pt)
        acc_t[...] = jnp.zeros_like(acc_t)
        acc_sp[...] = jnp.zeros_like(acc_sp)
        acc_ts[...] = jnp.zeros_like(acc_ts)

    coeffh = coeffh_ref[0]
    ptile = proto_ref[0].astype(jnp.bfloat16)
    rtile = jnp.concatenate(
        [tgt_ref[0].astype(jnp.bfloat16),
         jnp.ones((1, _K), jnp.bfloat16)], axis=0)
    xh = jnp.dot(coeffh, ptile,
                 preferred_element_type=jnp.float32).astype(jnp.bfloat16)
    tv = jnp.tanh(xh)
    spc = 2.0 * jnp.maximum(xh, 0.0) - jnp.log(1.0 + jnp.abs(tv))
    acc_pt[...] += jax.lax.dot_general(ptile, rtile, _DN,
                                       preferred_element_type=jnp.float32)
    acc_t[...] += jax.lax.dot_general(tv, rtile, _DN,
                                      preferred_element_type=jnp.float32)
    acc_sp[...] += jax.lax.dot_general(spc, rtile, _DN,
                                       preferred_element_type=jnp.float32)
    ones_row = jnp.ones((1, _K), jnp.bfloat16)
    acc_ts[...] += jax.lax.dot_general(ones_row, rtile, _DN,
                                       preferred_element_type=jnp.float32)

    @pl.when(k == _NK - 1)
    def _():
        xt = 2.0 * jnp.dot(coeffh_ref[0].astype(jnp.float32), acc_pt[:, :_G],
                           preferred_element_type=jnp.float32)
        ttm = acc_t[...]
        tsum = acc_ts[:, :_G]
        st = 0.5 * (tsum + ttm[:, :_G])
        ssum = 0.5 * (_THW + ttm[:, _G:])
        spsum = acc_sp[:, _G:] + _THW * _LN2
        cost_bce = (spsum - xt) * (1.0 / _THW)
        cost_dice = 1.0 - (2.0 * st + 1.0) / (ssum + tsum + 1.0)

        probs = jax.nn.sigmoid(cls_ref[0])
        labels = labels_ref[0]
        cls_iota = jax.lax.broadcasted_iota(jnp.int32, (_C, _G), 0)
        onehot = (cls_iota == labels).astype(jnp.float32)
        cost_class = -jnp.dot(probs, onehot, preferred_element_type=jnp.float32)

        bx = boxes_ref[0]
        tb = tgtbt_ref[0]
        ax0, ay0, ax1, ay1 = (bx[:, i:i + 1] for i in range(4))
        bx0, by0, bx1, by1 = (tb[i:i + 1, :] for i in range(4))
        l1 = (jnp.abs(ax0 - bx0) + jnp.abs(ay0 - by0)
              + jnp.abs(ax1 - bx1) + jnp.abs(ay1 - by1))
        area_a = (ax1 - ax0) * (ay1 - ay0)
        area_b = (bx1 - bx0) * (by1 - by0)
        iw = jnp.clip(jnp.minimum(ax1, bx1) - jnp.maximum(ax0, bx0), 0.0)
        ih = jnp.clip(jnp.minimum(ay1, by1) - jnp.maximum(ay0, by0), 0.0)
        inter = iw * ih
        union = area_a + area_b - inter
        iou = inter / (union + 1e-7)
        ew = jnp.maximum(ax1, bx1) - jnp.minimum(ax0, bx0)
        eh = jnp.maximum(ay1, by1) - jnp.minimum(ay0, by0)
        enc = jnp.clip(ew, 0.0) * jnp.clip(eh, 0.0)
        giou = iou - (enc - union) / (enc + 1e-7)
        cost_bbox = l1 + (1.0 - giou)

        cost = (_COST_CLASS * cost_class
                + _COST_DICE * (cost_bce + cost_dice)
                + _COST_BOX * cost_bbox)
        cost_ref[0] = cost
        qiota = jax.lax.broadcasted_iota(jnp.int32, (_Q, _G), 0)
        cmin = jnp.min(cost, axis=0, keepdims=True)
        match_ref[0] = jnp.min(jnp.where(cost == cmin, qiota, _Q), axis=0,
                               keepdims=True)


def kernel(cls, mask_coeff, proto, boxes, tgt_labels, tgt_masks, tgt_boxes):
    coeffh = (0.5 * mask_coeff).astype(jnp.bfloat16)
    proto2 = proto.reshape(_B, _M, _THW)
    tgt2 = tgt_masks.reshape(_B, _G, _THW)
    tgtbt = jnp.swapaxes(tgt_boxes, 1, 2)
    labels3 = tgt_labels.astype(jnp.int32).reshape(_B, 1, _G)
    cost, match3 = pl.pallas_call(
        _body,
        grid=(_B, _NK),
        in_specs=[
            pl.BlockSpec((1, _Q, _M), lambda b, k: (b, 0, 0)),
            pl.BlockSpec((1, _M, _K), lambda b, k: (b, 0, k)),
            pl.BlockSpec((1, _G, _K), lambda b, k: (b, 0, k)),
            pl.BlockSpec((1, _Q, _C), lambda b, k: (b, 0, 0)),
            pl.BlockSpec((1, _Q, 4), lambda b, k: (b, 0, 0)),
            pl.BlockSpec((1, 4, _G), lambda b, k: (b, 0, 0)),
            pl.BlockSpec((1, 1, _G), lambda b, k: (b, 0, 0)),
        ],
        out_specs=[
            pl.BlockSpec((1, _Q, _G), lambda b, k: (b, 0, 0)),
            pl.BlockSpec((1, 1, _G), lambda b, k: (b, 0, 0)),
        ],
        out_shape=[
            jax.ShapeDtypeStruct((_B, _Q, _G), jnp.float32),
            jax.ShapeDtypeStruct((_B, 1, _G), jnp.int32),
        ],
        scratch_shapes=[
            pltpu.VMEM((_M, _G1), jnp.float32),
            pltpu.VMEM((_Q, _G1), jnp.float32),
            pltpu.VMEM((_Q, _G1), jnp.float32),
            pltpu.VMEM((1, _G1), jnp.float32),
        ],
        compiler_params=pltpu.CompilerParams(
            dimension_semantics=("parallel", "arbitrary")),
    )(coeffh, proto2, tgt2, cls, boxes, tgtbt, labels3)
    return cost, match3[:, 0, :]
